# baseline (speedup 1.0000x reference)
import functools
import math

import numpy as np

import jax
import jax.numpy as jnp
from jax import lax
from jax.experimental import pallas as pl
from jax.experimental.pallas import tpu as pltpu
from jax.experimental.pallas import tpu_sc as plsc

VOCAB = 100000
EMB = 1024
FF = 256
MAX_LEN = 5000
B, S = 4, 2048
NTOK = B * S
SCALE = math.sqrt(EMB)

_NC, _NS = 2, 16
_NW = _NC * _NS
_QW = S // _NW
_CHUNK = 32
_QCH = _QW // _CHUNK


@functools.cache
def _make_gather_q(q):
    mesh = plsc.VectorSubcoreMesh(core_axis_name="c", subcore_axis_name="s")

    @functools.partial(
        pl.kernel,
        mesh=mesh,
        out_type=jax.ShapeDtypeStruct((S, EMB), jnp.float32),
        scratch_types=[
            pltpu.VMEM((_QCH, _CHUNK), jnp.int32),
            pltpu.VMEM((2, _CHUNK, EMB), jnp.float32),
            pltpu.SemaphoreType.DMA,
            pltpu.SemaphoreType.DMA,
            pltpu.SemaphoreType.DMA,
            pltpu.SemaphoreType.DMA,
        ],
    )
    def gather_k(table_hbm, idx_hbm, out_hbm, idx_v, rows_v, g0, g1, w0, w1):
        wid = lax.axis_index("s") * _NC + lax.axis_index("c")
        pltpu.sync_copy(idx_hbm.at[q, wid], idx_v)
        base = wid * _QW
        gsem = (g0, g1)
        wsem = (w0, w1)

        def out_slice(c):
            return out_hbm.at[pl.ds(base + c * _CHUNK, _CHUNK)]

        pltpu.async_copy(table_hbm.at[idx_v.at[0]], rows_v.at[0], gsem[0])
        for c in range(_QCH):
            s = c % 2
            if c + 1 < _QCH:
                s2 = (c + 1) % 2
                if c >= 1:
                    pltpu.make_async_copy(rows_v.at[s2], out_slice(c - 1),
                                          wsem[s2]).wait()
                pltpu.async_copy(table_hbm.at[idx_v.at[c + 1]], rows_v.at[s2],
                                 gsem[s2])
            pltpu.make_async_copy(table_hbm.at[idx_v.at[c]], rows_v.at[s],
                                  gsem[s]).wait()
            pltpu.async_copy(rows_v.at[s], out_slice(c), wsem[s])
        for c in (max(_QCH - 2, 0), _QCH - 1):
            pltpu.make_async_copy(rows_v.at[c % 2], out_slice(c),
                                  wsem[c % 2]).wait()

    return gather_k


_T = 512


def _adapter_body_first(emb_ref, w1_ref, b1_ref, w2_ref, b2_ref, sint_ref,
                        cost_ref, sinb_ref, cosb_ref, out_ref):
    e = emb_ref[...]
    h = jnp.maximum(
        jnp.dot(e, w1_ref[...], preferred_element_type=jnp.float32) + b1_ref[...],
        0.0,
    )
    o = e + jnp.dot(h, w2_ref[...], preferred_element_type=jnp.float32) + b2_ref[...]
    pe = sinb_ref[0] * cost_ref[...] + cosb_ref[0] * sint_ref[...]
    out_ref[0] = o * SCALE + pe


def _adapter_body_rest(prev_ref, emb_ref, w1_ref, b1_ref, w2_ref, b2_ref,
                       sint_ref, cost_ref, sinb_ref, cosb_ref, out_ref):
    _adapter_body_first(emb_ref, w1_ref, b1_ref, w2_ref, b2_ref, sint_ref,
                        cost_ref, sinb_ref, cosb_ref, out_ref)


def _adapter_q(q, prev, emb_q, W1, b1, W2, b2, sinT, cosT, sinB, cosB):
    grid = (S // _T,)
    specs = [
        pl.BlockSpec((_T, EMB), lambda p: (p, 0)),
        pl.BlockSpec((EMB, FF), lambda p: (0, 0)),
        pl.BlockSpec((1, FF), lambda p: (0, 0)),
        pl.BlockSpec((FF, EMB), lambda p: (0, 0)),
        pl.BlockSpec((1, EMB), lambda p: (0, 0)),
        pl.BlockSpec((_T, EMB), lambda p: (0, 0)),
        pl.BlockSpec((_T, EMB), lambda p: (0, 0)),
        pl.BlockSpec((1, 1, EMB), lambda p: (p, 0, 0)),
        pl.BlockSpec((1, 1, EMB), lambda p: (p, 0, 0)),
    ]
    args = [emb_q, W1, b1, W2, b2, sinT, cosT, sinB, cosB]
    if prev is None:
        body = _adapter_body_first
        aliases = {}
    else:
        body = _adapter_body_rest
        specs = [pl.BlockSpec(memory_space=pl.ANY)] + specs
        args = [prev] + args
        aliases = {0: 0}
    return pl.pallas_call(
        body,
        grid=grid,
        in_specs=specs,
        out_specs=pl.BlockSpec((1, _T, EMB), lambda p, _q=q: (_q, p, 0)),
        out_shape=jax.ShapeDtypeStruct((B, S, EMB), jnp.float32),
        input_output_aliases=aliases,
    )(*args)


def _pe_consts():
    half = np.exp(np.arange(0, EMB, 2, dtype=np.float64)
                  * (-(math.log(10000.0) / EMB)))
    divf = np.repeat(half, 2)
    phase = np.tile(np.array([0.0, math.pi / 2]), EMB // 2)
    t = np.arange(_T, dtype=np.float64)[:, None]
    sinT = np.sin(t * divf + phase).astype(np.float32)
    cosT = np.cos(t * divf + phase).astype(np.float32)
    p = np.arange(S // _T, dtype=np.float64)[:, None]
    sinB = np.sin(p * _T * divf).astype(np.float32)
    cosB = np.cos(p * _T * divf).astype(np.float32)
    return (jnp.asarray(sinT), jnp.asarray(cosT),
            jnp.asarray(sinB[:, None, :]), jnp.asarray(cosB[:, None, :]))


def kernel(x, table, W1, b1, W2, b2):
    idx4 = x.reshape(B, _NW, _QCH, _CHUNK).astype(jnp.int32)
    sinT, cosT, sinB, cosB = _pe_consts()
    b1r = b1.reshape(1, FF)
    b2r = b2.reshape(1, EMB)
    out = None
    for q in range(B):
        emb_q = _make_gather_q(q)(table, idx4)
        out = _adapter_q(q, out, emb_q, W1, b1r, W2, b2r,
                         sinT, cosT, sinB, cosB)
    return out


# speedup vs baseline: 1.0431x; 1.0431x over previous
import functools
import math

import numpy as np

import jax
import jax.numpy as jnp
from jax import lax
from jax.experimental import pallas as pl
from jax.experimental.pallas import tpu as pltpu
from jax.experimental.pallas import tpu_sc as plsc

VOCAB = 100000
EMB = 1024
FF = 256
MAX_LEN = 5000
B, S = 4, 2048
NTOK = B * S
SCALE = math.sqrt(EMB)

_NC, _NS = 2, 16
_NW = _NC * _NS
_QW = S // _NW
_CHUNK = 32
_QCH = _QW // _CHUNK


@functools.cache
def _make_gather_q(q):
    mesh = plsc.VectorSubcoreMesh(core_axis_name="c", subcore_axis_name="s")

    @functools.partial(
        pl.kernel,
        mesh=mesh,
        out_type=jax.ShapeDtypeStruct((S, EMB), jnp.float32),
        scratch_types=[
            pltpu.VMEM((_QCH, _CHUNK), jnp.int32),
            pltpu.VMEM((2, _CHUNK, EMB), jnp.float32),
            pltpu.SemaphoreType.DMA,
            pltpu.SemaphoreType.DMA,
            pltpu.SemaphoreType.DMA,
            pltpu.SemaphoreType.DMA,
        ],
    )
    def gather_k(table_hbm, idx_hbm, out_hbm, idx_v, rows_v, g0, g1, w0, w1):
        wid = lax.axis_index("s") * _NC + lax.axis_index("c")
        pltpu.sync_copy(idx_hbm.at[q, wid], idx_v)
        base = wid * _QW
        gsem = (g0, g1)
        wsem = (w0, w1)

        def out_slice(c):
            return out_hbm.at[pl.ds(base + c * _CHUNK, _CHUNK)]

        pltpu.async_copy(table_hbm.at[idx_v.at[0]], rows_v.at[0], gsem[0])
        for c in range(_QCH):
            s = c % 2
            if c + 1 < _QCH:
                s2 = (c + 1) % 2
                if c >= 1:
                    pltpu.make_async_copy(rows_v.at[s2], out_slice(c - 1),
                                          wsem[s2]).wait()
                pltpu.async_copy(table_hbm.at[idx_v.at[c + 1]], rows_v.at[s2],
                                 gsem[s2])
            pltpu.make_async_copy(table_hbm.at[idx_v.at[c]], rows_v.at[s],
                                  gsem[s]).wait()
            pltpu.async_copy(rows_v.at[s], out_slice(c), wsem[s])
        for c in (max(_QCH - 2, 0), _QCH - 1):
            pltpu.make_async_copy(rows_v.at[c % 2], out_slice(c),
                                  wsem[c % 2]).wait()

    return gather_k


_T = 512


_TP = 128


def _adapter_body_first(emb_ref, w1_ref, b1_ref, w2_ref, b2_ref, sint_ref,
                        cost_ref, sinb_ref, cosb_ref, out_ref):
    e = emb_ref[...]
    h = jnp.maximum(
        jnp.dot(e.astype(jnp.bfloat16), w1_ref[...],
                preferred_element_type=jnp.float32) + b1_ref[...],
        0.0,
    )
    o = e + jnp.dot(h.astype(jnp.bfloat16), w2_ref[...],
                    preferred_element_type=jnp.float32) + b2_ref[...]
    sinT = sint_ref[...]
    cosT = cost_ref[...]
    pe = jnp.concatenate(
        [sinb_ref[0, jj:jj + 1] * cosT + cosb_ref[0, jj:jj + 1] * sinT
         for jj in range(_T // _TP)], axis=0)
    out_ref[0] = o * SCALE + pe


def _adapter_body_rest(prev_ref, emb_ref, w1_ref, b1_ref, w2_ref, b2_ref,
                       sint_ref, cost_ref, sinb_ref, cosb_ref, out_ref):
    _adapter_body_first(emb_ref, w1_ref, b1_ref, w2_ref, b2_ref, sint_ref,
                        cost_ref, sinb_ref, cosb_ref, out_ref)


def _adapter_q(q, prev, emb_q, W1, b1, W2, b2, sinT, cosT, sinB, cosB):
    grid = (S // _T,)
    specs = [
        pl.BlockSpec((_T, EMB), lambda p: (p, 0)),
        pl.BlockSpec((EMB, FF), lambda p: (0, 0)),
        pl.BlockSpec((1, FF), lambda p: (0, 0)),
        pl.BlockSpec((FF, EMB), lambda p: (0, 0)),
        pl.BlockSpec((1, EMB), lambda p: (0, 0)),
        pl.BlockSpec((_TP, EMB), lambda p: (0, 0)),
        pl.BlockSpec((_TP, EMB), lambda p: (0, 0)),
        pl.BlockSpec((1, _T // _TP, EMB), lambda p: (p, 0, 0)),
        pl.BlockSpec((1, _T // _TP, EMB), lambda p: (p, 0, 0)),
    ]
    args = [emb_q, W1, b1, W2, b2, sinT, cosT, sinB, cosB]
    if prev is None:
        body = _adapter_body_first
        aliases = {}
    else:
        body = _adapter_body_rest
        specs = [pl.BlockSpec(memory_space=pl.ANY)] + specs
        args = [prev] + args
        aliases = {0: 0}
    return pl.pallas_call(
        body,
        grid=grid,
        in_specs=specs,
        out_specs=pl.BlockSpec((1, _T, EMB), lambda p, _q=q: (_q, p, 0)),
        out_shape=jax.ShapeDtypeStruct((B, S, EMB), jnp.float32),
        input_output_aliases=aliases,
    )(*args)


def _pe_consts():
    half = np.exp(np.arange(0, EMB, 2, dtype=np.float64)
                  * (-(math.log(10000.0) / EMB)))
    divf = np.repeat(half, 2)
    phase = np.tile(np.array([0.0, math.pi / 2]), EMB // 2)
    t = np.arange(_TP, dtype=np.float64)[:, None]
    sinT = np.sin(t * divf + phase).astype(np.float32)
    cosT = np.cos(t * divf + phase).astype(np.float32)
    p = np.arange(S // _TP, dtype=np.float64)[:, None]
    sinB = np.sin(p * _TP * divf).astype(np.float32)
    cosB = np.cos(p * _TP * divf).astype(np.float32)
    nb = _T // _TP
    return (jnp.asarray(sinT), jnp.asarray(cosT),
            jnp.asarray(sinB.reshape(S // _T, nb, EMB)),
            jnp.asarray(cosB.reshape(S // _T, nb, EMB)))


def kernel(x, table, W1, b1, W2, b2):
    idx4 = x.reshape(B, _NW, _QCH, _CHUNK).astype(jnp.int32)
    sinT, cosT, sinB, cosB = _pe_consts()
    W1 = W1.astype(jnp.bfloat16)
    W2 = W2.astype(jnp.bfloat16)
    b1r = b1.reshape(1, FF)
    b2r = b2.reshape(1, EMB)
    out = None
    for q in range(B):
        emb_q = _make_gather_q(q)(table, idx4)
        out = _adapter_q(q, out, emb_q, W1, b1r, W2, b2r,
                         sinT, cosT, sinB, cosB)
    return out
